# baseline (speedup 1.0000x reference)
import functools

import jax
import jax.numpy as jnp
from jax import lax
from jax.experimental import pallas as pl
from jax.experimental.pallas import tpu as pltpu
from jax.experimental.pallas import tpu_sc as plsc

N_NODES = 10000
N_EDGES = 320000
IN_DIM = 128
NUM_HEADS = 8
OUT_DIM = 16
HD = NUM_HEADS * OUT_DIM

NW = 32
EPW = N_EDGES // NW
CHUNK = 80
NCHUNK = EPW // CHUNK
NPAD = 10240
EXPORT_ROWS = 128
NPT = NPAD // 16
NSTAGE = NPT // EXPORT_ROWS

_mesh = plsc.VectorSubcoreMesh(core_axis_name="c", subcore_axis_name="s",
                               num_cores=2, num_subcores=16)


def _qkv_body(h_ref, wq_ref, wk_ref, wv_ref, q_ref, k_ref, v_ref):
    hv = h_ref[...]
    q_ref[...] = jnp.dot(hv, wq_ref[...], preferred_element_type=jnp.float32, precision=lax.Precision.HIGHEST)
    k_ref[...] = jnp.dot(hv, wk_ref[...], preferred_element_type=jnp.float32, precision=lax.Precision.HIGHEST)
    v_ref[...] = jnp.dot(hv, wv_ref[...], preferred_element_type=jnp.float32, precision=lax.Precision.HIGHEST)


def _qkv(h, WQ, WK, WV):
    n = h.shape[0]
    out = jax.ShapeDtypeStruct((n, HD), jnp.float32)
    return pl.pallas_call(
        _qkv_body,
        out_shape=(out, out, out),
    )(h, WQ, WK, WV)


def _score_body(k_hbm, q_hbm, src_hbm, dst_hbm, score_hbm,
                srcbuf, dstbuf, krows, qrows, sbuf, sem0, sem1):
    wid = lax.axis_index("s") * 2 + lax.axis_index("c")
    iota16 = lax.iota(jnp.int32, 16)

    def chunk_body(i, _):
        base = wid * jnp.int32(EPW) + i * jnp.int32(CHUNK)
        pltpu.sync_copy(src_hbm.at[pl.ds(base, CHUNK)], srcbuf)
        pltpu.sync_copy(dst_hbm.at[pl.ds(base, CHUNK)], dstbuf)
        cp0 = pltpu.async_copy(k_hbm.at[srcbuf], krows, sem0)
        cp1 = pltpu.async_copy(q_hbm.at[dstbuf], qrows, sem1)
        cp0.wait()
        cp1.wait()

        def group_body(g, _):
            rows = g * jnp.int32(16) + iota16
            for h in range(NUM_HEADS):
                acc = jnp.zeros((16,), jnp.float32)
                for d in range(OUT_DIM):
                    cols = jnp.full((16,), h * OUT_DIM + d, jnp.int32)
                    kv = plsc.load_gather(krows, [rows, cols])
                    qv = plsc.load_gather(qrows, [rows, cols])
                    acc = acc + kv * qv
                plsc.store_scatter(sbuf, [rows, jnp.full((16,), h, jnp.int32)], acc)
            return jnp.int32(0)

        lax.fori_loop(jnp.int32(0), jnp.int32(CHUNK // 16), group_body, jnp.int32(0))
        pltpu.sync_copy(sbuf, score_hbm.at[pl.ds(base, CHUNK)])
        return jnp.int32(0)

    lax.fori_loop(jnp.int32(0), jnp.int32(NCHUNK), chunk_body, jnp.int32(0))


def _score(K, Q, src, dst):
    return pl.kernel(
        _score_body,
        out_type=jax.ShapeDtypeStruct((N_EDGES, NUM_HEADS), jnp.float32),
        mesh=_mesh,
        compiler_params=pltpu.CompilerParams(needs_layout_passes=False),
        scratch_types=[
            pltpu.VMEM((CHUNK,), jnp.int32),
            pltpu.VMEM((CHUNK,), jnp.int32),
            pltpu.VMEM((CHUNK, HD), jnp.float32),
            pltpu.VMEM((CHUNK, HD), jnp.float32),
            pltpu.VMEM((CHUNK, NUM_HEADS), jnp.float32),
            pltpu.SemaphoreType.DMA,
            pltpu.SemaphoreType.DMA,
        ],
    )(K, Q, src, dst)


def _eout_body(e_ref, we_ref, sc_ref, eo_ref, p_ref):
    proj = jnp.dot(e_ref[...], we_ref[...], preferred_element_type=jnp.float32, precision=lax.Precision.HIGHEST)
    heads = lax.broadcasted_iota(jnp.int32, (NUM_HEADS, HD), 0)
    lanes = lax.broadcasted_iota(jnp.int32, (NUM_HEADS, HD), 1)
    expand = (lanes // OUT_DIM == heads).astype(jnp.float32)
    scb = jnp.dot(sc_ref[...], expand, preferred_element_type=jnp.float32, precision=lax.Precision.HIGHEST)
    eo = proj + scb * (1.0 / 4.0)
    eo_ref[...] = eo
    p_ref[...] = jnp.exp(jnp.minimum(eo, 60.0))


def _eout(e, We, score):
    rows = 4000
    grid = (N_EDGES // rows,)
    out = jax.ShapeDtypeStruct((N_EDGES, HD), jnp.float32)
    return pl.pallas_call(
        _eout_body,
        grid=grid,
        in_specs=[
            pl.BlockSpec((rows, IN_DIM), lambda i: (i, jnp.int32(0))),
            pl.BlockSpec((IN_DIM, HD), lambda i: (jnp.int32(0), jnp.int32(0))),
            pl.BlockSpec((rows, NUM_HEADS), lambda i: (i, jnp.int32(0))),
        ],
        out_specs=(
            pl.BlockSpec((rows, HD), lambda i: (i, jnp.int32(0))),
            pl.BlockSpec((rows, HD), lambda i: (i, jnp.int32(0))),
        ),
        out_shape=(out, out),
    )(e, We, score)


def _agg_body(p_hbm, v_hbm, src_hbm, dst_hbm, den_hbm, wv_hbm,
              srcbuf, dstbuf, prows, vrows, stage, acc_shared, sem0):
    cid = lax.axis_index("c")
    sid = lax.axis_index("s")
    wid = sid * 2 + cid

    def fill_zeros(_):
        def zrow(r, _):
            for k in range(HD // 16):
                stage[r, pl.ds(k * 16, 16)] = jnp.zeros((16,), jnp.float32)
            return jnp.int32(0)
        lax.fori_loop(jnp.int32(0), jnp.int32(EXPORT_ROWS), zrow, jnp.int32(0))

    def zero_shared(_):
        for t in range(NSTAGE):
            row0 = sid * jnp.int32(NPT) + jnp.int32(t * EXPORT_ROWS)
            pltpu.sync_copy(stage, acc_shared.at[pl.ds(row0, EXPORT_ROWS)])

    def export(out_hbm):
        for t in range(NSTAGE):
            row0 = sid * jnp.int32(NPT) + jnp.int32(t * EXPORT_ROWS)
            pltpu.sync_copy(acc_shared.at[pl.ds(row0, EXPORT_ROWS)], stage)
            pltpu.sync_copy(stage, out_hbm.at[cid, pl.ds(row0, EXPORT_ROWS)])

    fill_zeros(None)
    zero_shared(None)
    plsc.subcore_barrier()

    def chunk_a(i, _):
        base = wid * jnp.int32(EPW) + i * jnp.int32(CHUNK)
        pltpu.sync_copy(dst_hbm.at[pl.ds(base, CHUNK)], dstbuf)
        pltpu.sync_copy(p_hbm.at[pl.ds(base, CHUNK)], prows)
        pltpu.sync_copy(prows, acc_shared.at[dstbuf], add=True)
        return jnp.int32(0)

    lax.fori_loop(jnp.int32(0), jnp.int32(NCHUNK), chunk_a, jnp.int32(0))
    plsc.subcore_barrier()
    export(den_hbm)
    plsc.subcore_barrier()

    fill_zeros(None)
    zero_shared(None)
    plsc.subcore_barrier()

    def chunk_b(i, _):
        base = wid * jnp.int32(EPW) + i * jnp.int32(CHUNK)
        pltpu.sync_copy(dst_hbm.at[pl.ds(base, CHUNK)], dstbuf)
        pltpu.sync_copy(src_hbm.at[pl.ds(base, CHUNK)], srcbuf)
        pltpu.sync_copy(p_hbm.at[pl.ds(base, CHUNK)], prows)
        pltpu.async_copy(v_hbm.at[srcbuf], vrows, sem0).wait()

        def mrow(r, _):
            for k in range(HD // 16):
                sl = pl.ds(k * 16, 16)
                prows[r, sl] = prows[r, sl] * vrows[r, sl]
            return jnp.int32(0)

        lax.fori_loop(jnp.int32(0), jnp.int32(CHUNK), mrow, jnp.int32(0))
        pltpu.sync_copy(prows, acc_shared.at[dstbuf], add=True)
        return jnp.int32(0)

    lax.fori_loop(jnp.int32(0), jnp.int32(NCHUNK), chunk_b, jnp.int32(0))
    plsc.subcore_barrier()
    export(wv_hbm)


def _aggregate(P, V, src, dst):
    out = jax.ShapeDtypeStruct((2, NPAD, HD), jnp.float32)
    return pl.kernel(
        _agg_body,
        out_type=(out, out),
        mesh=_mesh,
        scratch_types=[
            pltpu.VMEM((CHUNK,), jnp.int32),
            pltpu.VMEM((CHUNK,), jnp.int32),
            pltpu.VMEM((CHUNK, HD), jnp.float32),
            pltpu.VMEM((CHUNK, HD), jnp.float32),
            pltpu.VMEM((EXPORT_ROWS, HD), jnp.float32),
            pltpu.VMEM_SHARED((NPAD, HD), jnp.float32),
            pltpu.SemaphoreType.DMA,
        ],
    )(P, V, src, dst)


def _div_body(wv_ref, den_ref, out_ref):
    wv = wv_ref[0, :N_NODES] + wv_ref[1, :N_NODES]
    den = den_ref[0, :N_NODES] + den_ref[1, :N_NODES]
    out_ref[...] = wv / den


def _divide(wv_parts, den_parts):
    return pl.pallas_call(
        _div_body,
        out_shape=jax.ShapeDtypeStruct((N_NODES, HD), jnp.float32),
    )(wv_parts, den_parts)


def _f32_to_f64(x):
    ub = lax.bitcast_convert_type(x, jnp.uint32)
    sign = ub >> 31
    exp8 = (ub >> 23) & jnp.uint32(0xFF)
    mant = ub & jnp.uint32(0x7FFFFF)
    nz = exp8 > 0
    exp11 = jnp.where(nz, exp8 + jnp.uint32(1023 - 127), jnp.uint32(0))
    hi = (sign << 31) | (exp11 << 20) | jnp.where(nz, mant >> 3, jnp.uint32(0))
    lo = jnp.where(nz, (mant & jnp.uint32(7)) << 29, jnp.uint32(0))
    w64 = (hi.astype(jnp.uint64) << 32) | lo.astype(jnp.uint64)
    return lax.bitcast_convert_type(w64, jnp.float64)


@jax.jit
def kernel(h, e, edge_index, WQ, WK, WV, We):
    h = h.astype(jnp.float32)
    e = e.astype(jnp.float32)
    src = edge_index[0].astype(jnp.int32)
    dst = edge_index[1].astype(jnp.int32)

    Q, K, V = _qkv(h, WQ.astype(jnp.float32), WK.astype(jnp.float32),
                   WV.astype(jnp.float32))
    score = _score(K, Q, src, dst)
    e_out, P = _eout(e, We.astype(jnp.float32), score)
    den_parts, wv_parts = _aggregate(P, V, src, dst)
    wv = _divide(wv_parts, den_parts)
    return (_f32_to_f64(wv.reshape(N_NODES, NUM_HEADS, OUT_DIM)),
            _f32_to_f64(e_out.reshape(N_EDGES, NUM_HEADS, OUT_DIM)))


# speedup vs baseline: 1.0137x; 1.0137x over previous
import functools

import jax
import jax.numpy as jnp
from jax import lax
from jax.experimental import pallas as pl
from jax.experimental.pallas import tpu as pltpu
from jax.experimental.pallas import tpu_sc as plsc

N_NODES = 10000
N_EDGES = 320000
IN_DIM = 128
NUM_HEADS = 8
OUT_DIM = 16
HD = NUM_HEADS * OUT_DIM

NW = 32
EPW = N_EDGES // NW
CHUNK = 80
NCHUNK = EPW // CHUNK
NPAD = 10240
EXPORT_ROWS = 128
NPT = NPAD // 16
NSTAGE = NPT // EXPORT_ROWS

_mesh = plsc.VectorSubcoreMesh(core_axis_name="c", subcore_axis_name="s",
                               num_cores=2, num_subcores=16)


def _qkv_body(h_ref, wq_ref, wk_ref, wv_ref, q_ref, k_ref, v_ref):
    hv = h_ref[...]
    q_ref[...] = jnp.dot(hv, wq_ref[...], preferred_element_type=jnp.float32, precision=lax.Precision.HIGHEST)
    k_ref[...] = jnp.dot(hv, wk_ref[...], preferred_element_type=jnp.float32, precision=lax.Precision.HIGHEST)
    v_ref[...] = jnp.dot(hv, wv_ref[...], preferred_element_type=jnp.float32, precision=lax.Precision.HIGHEST)


def _qkv(h, WQ, WK, WV):
    n = h.shape[0]
    rows = 2000
    out = jax.ShapeDtypeStruct((n, HD), jnp.float32)
    wspec = pl.BlockSpec((IN_DIM, HD), lambda i: (jnp.int32(0), jnp.int32(0)))
    rspec = pl.BlockSpec((rows, HD), lambda i: (i, jnp.int32(0)))
    return pl.pallas_call(
        _qkv_body,
        grid=(n // rows,),
        in_specs=[pl.BlockSpec((rows, IN_DIM), lambda i: (i, jnp.int32(0))),
                  wspec, wspec, wspec],
        out_specs=(rspec, rspec, rspec),
        out_shape=(out, out, out),
    )(h, WQ, WK, WV)


def _score_body(k_hbm, q_hbm, src_hbm, dst_hbm, score_hbm,
                srcbuf, dstbuf, krows, qrows, sbuf, sem0, sem1):
    wid = lax.axis_index("s") * 2 + lax.axis_index("c")
    iota16 = lax.iota(jnp.int32, 16)

    def chunk_body(i, _):
        base = wid * jnp.int32(EPW) + i * jnp.int32(CHUNK)
        pltpu.sync_copy(src_hbm.at[pl.ds(base, CHUNK)], srcbuf)
        pltpu.sync_copy(dst_hbm.at[pl.ds(base, CHUNK)], dstbuf)
        cp0 = pltpu.async_copy(k_hbm.at[srcbuf], krows, sem0)
        cp1 = pltpu.async_copy(q_hbm.at[dstbuf], qrows, sem1)
        cp0.wait()
        cp1.wait()

        def group_body(g, _):
            rows = g * jnp.int32(16) + iota16
            for h in range(NUM_HEADS):
                acc = jnp.zeros((16,), jnp.float32)
                for d in range(OUT_DIM):
                    cols = jnp.full((16,), h * OUT_DIM + d, jnp.int32)
                    kv = plsc.load_gather(krows, [rows, cols])
                    qv = plsc.load_gather(qrows, [rows, cols])
                    acc = acc + kv * qv
                plsc.store_scatter(sbuf, [rows, jnp.full((16,), h, jnp.int32)], acc)
            return jnp.int32(0)

        lax.fori_loop(jnp.int32(0), jnp.int32(CHUNK // 16), group_body, jnp.int32(0))
        pltpu.sync_copy(sbuf, score_hbm.at[pl.ds(base, CHUNK)])
        return jnp.int32(0)

    lax.fori_loop(jnp.int32(0), jnp.int32(NCHUNK), chunk_body, jnp.int32(0))


def _score(K, Q, src, dst):
    return pl.kernel(
        _score_body,
        out_type=jax.ShapeDtypeStruct((N_EDGES, NUM_HEADS), jnp.float32),
        mesh=_mesh,
        compiler_params=pltpu.CompilerParams(needs_layout_passes=False),
        scratch_types=[
            pltpu.VMEM((CHUNK,), jnp.int32),
            pltpu.VMEM((CHUNK,), jnp.int32),
            pltpu.VMEM((CHUNK, HD), jnp.float32),
            pltpu.VMEM((CHUNK, HD), jnp.float32),
            pltpu.VMEM((CHUNK, NUM_HEADS), jnp.float32),
            pltpu.SemaphoreType.DMA,
            pltpu.SemaphoreType.DMA,
        ],
    )(K, Q, src, dst)


def _f64_words(vals, rows):
    bits2 = lax.bitcast_convert_type(vals, jnp.int32)
    dupa = lax.broadcasted_iota(jnp.int32, (HD, 2 * HD), 0)
    dupb = lax.broadcasted_iota(jnp.int32, (HD, 2 * HD), 1)
    dup = (dupb // 2 == dupa).astype(jnp.float32)
    v2 = jnp.dot(vals, dup, preferred_element_type=jnp.float32,
                 precision=lax.Precision.HIGHEST)
    bits = lax.bitcast_convert_type(v2, jnp.int32)
    odd = (lax.broadcasted_iota(jnp.int32, (rows, 2 * HD), 1) & 1) == 1
    mag = bits & jnp.int32(0x7FFFFFFF)
    sgn = bits & jnp.int32(-2147483648)
    hi = sgn | ((mag >> 3) + jnp.int32(0x38000000))
    hi = jnp.where(mag == 0, sgn, hi)
    lo = bits << 29
    return jnp.where(odd, hi, lo)


def _eout_body(e_ref, we_ref, sc_ref, pair_ref, p_ref):
    rows = p_ref.shape[0]
    proj = jnp.dot(e_ref[...], we_ref[...], preferred_element_type=jnp.float32, precision=lax.Precision.HIGHEST)
    heads = lax.broadcasted_iota(jnp.int32, (NUM_HEADS, HD), 0)
    lanes = lax.broadcasted_iota(jnp.int32, (NUM_HEADS, HD), 1)
    expand = (lanes // OUT_DIM == heads).astype(jnp.float32)
    scb = jnp.dot(sc_ref[...], expand, preferred_element_type=jnp.float32, precision=lax.Precision.HIGHEST)
    eo = proj + scb * (1.0 / 4.0)
    pair_ref[...] = _f64_words(eo, rows)
    p_ref[...] = jnp.exp(jnp.minimum(eo, 60.0))


def _eout(e, We, score):
    rows = 4000
    grid = (N_EDGES // rows,)
    return pl.pallas_call(
        _eout_body,
        grid=grid,
        in_specs=[
            pl.BlockSpec((rows, IN_DIM), lambda i: (i, jnp.int32(0))),
            pl.BlockSpec((IN_DIM, HD), lambda i: (jnp.int32(0), jnp.int32(0))),
            pl.BlockSpec((rows, NUM_HEADS), lambda i: (i, jnp.int32(0))),
        ],
        out_specs=(
            pl.BlockSpec((rows, 2 * HD), lambda i: (i, jnp.int32(0))),
            pl.BlockSpec((rows, HD), lambda i: (i, jnp.int32(0))),
        ),
        out_shape=(jax.ShapeDtypeStruct((N_EDGES, 2 * HD), jnp.int32),
                   jax.ShapeDtypeStruct((N_EDGES, HD), jnp.float32)),
    )(e, We, score)


def _agg_body(p_hbm, v_hbm, src_hbm, dst_hbm, den_hbm, wv_hbm,
              srcbuf, dstbuf, prows, vrows, stage, acc_shared, sem0):
    cid = lax.axis_index("c")
    sid = lax.axis_index("s")
    wid = sid * 2 + cid

    def fill_zeros(_):
        def zrow(r, _):
            for k in range(HD // 16):
                stage[r, pl.ds(k * 16, 16)] = jnp.zeros((16,), jnp.float32)
            return jnp.int32(0)
        lax.fori_loop(jnp.int32(0), jnp.int32(EXPORT_ROWS), zrow, jnp.int32(0))

    def zero_shared(_):
        for t in range(NSTAGE):
            row0 = sid * jnp.int32(NPT) + jnp.int32(t * EXPORT_ROWS)
            pltpu.sync_copy(stage, acc_shared.at[pl.ds(row0, EXPORT_ROWS)])

    def export(out_hbm):
        for t in range(NSTAGE):
            row0 = sid * jnp.int32(NPT) + jnp.int32(t * EXPORT_ROWS)
            pltpu.sync_copy(acc_shared.at[pl.ds(row0, EXPORT_ROWS)], stage)
            pltpu.sync_copy(stage, out_hbm.at[cid, pl.ds(row0, EXPORT_ROWS)])

    fill_zeros(None)
    zero_shared(None)
    plsc.subcore_barrier()

    def chunk_a(i, _):
        base = wid * jnp.int32(EPW) + i * jnp.int32(CHUNK)
        pltpu.sync_copy(dst_hbm.at[pl.ds(base, CHUNK)], dstbuf)
        pltpu.sync_copy(p_hbm.at[pl.ds(base, CHUNK)], prows)
        pltpu.sync_copy(prows, acc_shared.at[dstbuf], add=True)
        return jnp.int32(0)

    lax.fori_loop(jnp.int32(0), jnp.int32(NCHUNK), chunk_a, jnp.int32(0))
    plsc.subcore_barrier()
    export(den_hbm)
    plsc.subcore_barrier()

    fill_zeros(None)
    zero_shared(None)
    plsc.subcore_barrier()

    def chunk_b(i, _):
        base = wid * jnp.int32(EPW) + i * jnp.int32(CHUNK)
        pltpu.sync_copy(dst_hbm.at[pl.ds(base, CHUNK)], dstbuf)
        pltpu.sync_copy(src_hbm.at[pl.ds(base, CHUNK)], srcbuf)
        pltpu.sync_copy(p_hbm.at[pl.ds(base, CHUNK)], prows)
        pltpu.async_copy(v_hbm.at[srcbuf], vrows, sem0).wait()

        def mrow(r, _):
            for k in range(HD // 16):
                sl = pl.ds(k * 16, 16)
                prows[r, sl] = prows[r, sl] * vrows[r, sl]
            return jnp.int32(0)

        lax.fori_loop(jnp.int32(0), jnp.int32(CHUNK), mrow, jnp.int32(0))
        pltpu.sync_copy(prows, acc_shared.at[dstbuf], add=True)
        return jnp.int32(0)

    lax.fori_loop(jnp.int32(0), jnp.int32(NCHUNK), chunk_b, jnp.int32(0))
    plsc.subcore_barrier()
    export(wv_hbm)


def _aggregate(P, V, src, dst):
    out = jax.ShapeDtypeStruct((2, NPAD, HD), jnp.float32)
    return pl.kernel(
        _agg_body,
        out_type=(out, out),
        mesh=_mesh,
        scratch_types=[
            pltpu.VMEM((CHUNK,), jnp.int32),
            pltpu.VMEM((CHUNK,), jnp.int32),
            pltpu.VMEM((CHUNK, HD), jnp.float32),
            pltpu.VMEM((CHUNK, HD), jnp.float32),
            pltpu.VMEM((EXPORT_ROWS, HD), jnp.float32),
            pltpu.VMEM_SHARED((NPAD, HD), jnp.float32),
            pltpu.SemaphoreType.DMA,
        ],
    )(P, V, src, dst)


def _div_body(wv_ref, den_ref, out_ref):
    rows = out_ref.shape[0]
    wv = wv_ref[0] + wv_ref[1]
    den = den_ref[0] + den_ref[1]
    out_ref[...] = _f64_words(wv / den, rows)


def _divide(wv_parts, den_parts):
    rows = 2000
    inspec = pl.BlockSpec((2, rows, HD), lambda i: (jnp.int32(0), i, jnp.int32(0)))
    return pl.pallas_call(
        _div_body,
        grid=(N_NODES // rows,),
        in_specs=[inspec, inspec],
        out_specs=pl.BlockSpec((rows, 2 * HD), lambda i: (i, jnp.int32(0))),
        out_shape=jax.ShapeDtypeStruct((N_NODES, 2 * HD), jnp.int32),
    )(wv_parts, den_parts)


@jax.jit
def kernel(h, e, edge_index, WQ, WK, WV, We):
    h = h.astype(jnp.float32)
    e = e.astype(jnp.float32)
    src = edge_index[0].astype(jnp.int32)
    dst = edge_index[1].astype(jnp.int32)

    Q, K, V = _qkv(h, WQ.astype(jnp.float32), WK.astype(jnp.float32),
                   WV.astype(jnp.float32))
    score = _score(K, Q, src, dst)
    eo_pair, P = _eout(e, We.astype(jnp.float32), score)
    den_parts, wv_parts = _aggregate(P, V, src, dst)
    wv_pair = _divide(wv_parts, den_parts)
    e_out64 = lax.bitcast_convert_type(
        eo_pair.reshape(N_EDGES, HD, 2), jnp.float64)
    wv64 = lax.bitcast_convert_type(
        wv_pair.reshape(N_NODES, HD, 2), jnp.float64)
    return (wv64.reshape(N_NODES, NUM_HEADS, OUT_DIM),
            e_out64.reshape(N_EDGES, NUM_HEADS, OUT_DIM))


# speedup vs baseline: 1.6388x; 1.6166x over previous
import jax
import jax.numpy as jnp
from jax import lax
from jax.experimental import pallas as pl
from jax.experimental.pallas import tpu as pltpu
from jax.experimental.pallas import tpu_sc as plsc

N_NODES = 10000
N_EDGES = 320000
IN_DIM = 128
NUM_HEADS = 8
OUT_DIM = 16
HD = NUM_HEADS * OUT_DIM

NW = 32
EPW = N_EDGES // NW
CHUNK = 80
NCHUNK = EPW // CHUNK
NPAD = 10240
EXPORT_ROWS = 32
NPT = NPAD // 16
NSTAGE = NPT // EXPORT_ROWS

_mesh = plsc.VectorSubcoreMesh(core_axis_name="c", subcore_axis_name="s",
                               num_cores=2, num_subcores=16)


def _qkv_body(h_ref, wq_ref, wk_ref, wv_ref, q_ref, k_ref, v_ref):
    hv = h_ref[...]
    kw = dict(preferred_element_type=jnp.float32, precision=lax.Precision.HIGHEST)
    q_ref[...] = jnp.dot(hv, wq_ref[...], **kw)
    k_ref[...] = jnp.dot(hv, wk_ref[...], **kw)
    v_ref[...] = jnp.dot(hv, wv_ref[...], **kw)


def _qkv(h, WQ, WK, WV):
    n = h.shape[0]
    rows = 2000
    out = jax.ShapeDtypeStruct((n, HD), jnp.float32)
    wspec = pl.BlockSpec((IN_DIM, HD), lambda i: (jnp.int32(0), jnp.int32(0)))
    rspec = pl.BlockSpec((rows, HD), lambda i: (i, jnp.int32(0)))
    return pl.pallas_call(
        _qkv_body,
        grid=(n // rows,),
        in_specs=[pl.BlockSpec((rows, IN_DIM), lambda i: (i, jnp.int32(0))),
                  wspec, wspec, wspec],
        out_specs=(rspec, rspec, rspec),
        out_shape=(out, out, out),
    )(h, WQ, WK, WV)


def _score_body(k_hbm, q_hbm, src2_hbm, dst2_hbm, score_hbm,
                srcb, dstb, kr0, qr0, kr1, qr1, sbuf,
                semk0, semq0, semk1, semq1):
    wid = lax.axis_index("s") * 2 + lax.axis_index("c")
    iota16 = lax.iota(jnp.int32, 16)
    pltpu.sync_copy(src2_hbm.at[wid], srcb)
    pltpu.sync_copy(dst2_hbm.at[wid], dstb)

    def start(i, kr, qr, semk, semq):
        pltpu.async_copy(k_hbm.at[srcb.at[i]], kr, semk)
        pltpu.async_copy(q_hbm.at[dstb.at[i]], qr, semq)

    start(jnp.int32(0), kr0, qr0, semk0, semq0)
    start(jnp.int32(1), kr1, qr1, semk1, semq1)

    def chunk_body(i, _):
        base = wid * jnp.int32(EPW) + i * jnp.int32(CHUNK)

        def do(kr, qr, semk, semq):
            pltpu.make_async_copy(k_hbm.at[srcb.at[i]], kr, semk).wait()
            pltpu.make_async_copy(q_hbm.at[dstb.at[i]], qr, semq).wait()

            def group_body(g, _):
                rows = g * jnp.int32(16) + iota16
                for h in range(NUM_HEADS):
                    acc0 = jnp.zeros((16,), jnp.float32)
                    acc1 = jnp.zeros((16,), jnp.float32)
                    for d in range(0, OUT_DIM, 2):
                        c0 = jnp.full((16,), h * OUT_DIM + d, jnp.int32)
                        c1 = jnp.full((16,), h * OUT_DIM + d + 1, jnp.int32)
                        acc0 = acc0 + (plsc.load_gather(kr, [rows, c0])
                                       * plsc.load_gather(qr, [rows, c0]))
                        acc1 = acc1 + (plsc.load_gather(kr, [rows, c1])
                                       * plsc.load_gather(qr, [rows, c1]))
                    plsc.store_scatter(
                        sbuf, [rows, jnp.full((16,), h, jnp.int32)], acc0 + acc1)
                return jnp.int32(0)

            lax.fori_loop(jnp.int32(0), jnp.int32(CHUNK // 16), group_body,
                          jnp.int32(0))

            @pl.when(i + 2 < NCHUNK)
            def _():
                start(i + jnp.int32(2), kr, qr, semk, semq)

        @pl.when(i % 2 == 0)
        def _():
            do(kr0, qr0, semk0, semq0)

        @pl.when(i % 2 == 1)
        def _():
            do(kr1, qr1, semk1, semq1)

        pltpu.sync_copy(sbuf, score_hbm.at[pl.ds(base, CHUNK)])
        return jnp.int32(0)

    lax.fori_loop(jnp.int32(0), jnp.int32(NCHUNK), chunk_body, jnp.int32(0))


def _score(K, Q, src2, dst2):
    return pl.kernel(
        _score_body,
        out_type=jax.ShapeDtypeStruct((N_EDGES, NUM_HEADS), jnp.float32),
        mesh=_mesh,
        compiler_params=pltpu.CompilerParams(needs_layout_passes=False),
        scratch_types=[
            pltpu.VMEM((NCHUNK, CHUNK), jnp.int32),
            pltpu.VMEM((NCHUNK, CHUNK), jnp.int32),
            pltpu.VMEM((CHUNK, HD), jnp.float32),
            pltpu.VMEM((CHUNK, HD), jnp.float32),
            pltpu.VMEM((CHUNK, HD), jnp.float32),
            pltpu.VMEM((CHUNK, HD), jnp.float32),
            pltpu.VMEM((CHUNK, NUM_HEADS), jnp.float32),
            pltpu.SemaphoreType.DMA,
            pltpu.SemaphoreType.DMA,
            pltpu.SemaphoreType.DMA,
            pltpu.SemaphoreType.DMA,
        ],
    )(K, Q, src2, dst2)


def _eout_body(e_ref, we_ref, sc_ref, eo_ref, p_ref):
    kw = dict(preferred_element_type=jnp.float32, precision=lax.Precision.HIGHEST)
    proj = jnp.dot(e_ref[...], we_ref[...], **kw)
    heads = lax.broadcasted_iota(jnp.int32, (NUM_HEADS, HD), 0)
    lanes = lax.broadcasted_iota(jnp.int32, (NUM_HEADS, HD), 1)
    expand = (lanes // OUT_DIM == heads).astype(jnp.float32)
    scb = jnp.dot(sc_ref[...], expand, **kw)
    eo = proj + scb * (1.0 / 4.0)
    eo_ref[...] = eo
    p_ref[...] = jnp.exp(jnp.minimum(eo, 60.0))


def _eout(e, We, score):
    rows = 4000
    grid = (N_EDGES // rows,)
    out = jax.ShapeDtypeStruct((N_EDGES, HD), jnp.float32)
    return pl.pallas_call(
        _eout_body,
        grid=grid,
        in_specs=[
            pl.BlockSpec((rows, IN_DIM), lambda i: (i, jnp.int32(0))),
            pl.BlockSpec((IN_DIM, HD), lambda i: (jnp.int32(0), jnp.int32(0))),
            pl.BlockSpec((rows, NUM_HEADS), lambda i: (i, jnp.int32(0))),
        ],
        out_specs=(
            pl.BlockSpec((rows, HD), lambda i: (i, jnp.int32(0))),
            pl.BlockSpec((rows, HD), lambda i: (i, jnp.int32(0))),
        ),
        out_shape=(out, out),
    )(e, We, score)


def _agg_body(p_hbm, v_hbm, src1_hbm, dst1_hbm, den_hbm, wv_hbm,
              s0, s1, d0, d1, p0, p1, v0, v1, stage, acc_shared,
              semp0, semp1, semv0, semv1, semd0, semd1, sems0, sems1):
    cid = lax.axis_index("c")
    sid = lax.axis_index("s")
    wid = sid * 2 + cid

    def fill_zeros():
        def zrow(r, _):
            for k in range(HD // 16):
                stage[r, pl.ds(k * 16, 16)] = jnp.zeros((16,), jnp.float32)
            return jnp.int32(0)
        lax.fori_loop(jnp.int32(0), jnp.int32(EXPORT_ROWS), zrow, jnp.int32(0))

    def zero_shared():
        for t in range(NSTAGE):
            r0 = sid * jnp.int32(NPT) + jnp.int32(t * EXPORT_ROWS)
            pltpu.sync_copy(stage, acc_shared.at[pl.ds(r0, EXPORT_ROWS)])

    def export(out_hbm):
        for t in range(NSTAGE):
            r0 = sid * jnp.int32(NPT) + jnp.int32(t * EXPORT_ROWS)
            pltpu.sync_copy(acc_shared.at[pl.ds(r0, EXPORT_ROWS)], stage)
            pltpu.sync_copy(stage, out_hbm.at[cid, pl.ds(r0, EXPORT_ROWS)])

    def chunk_base(i):
        return wid * jnp.int32(EPW) + i * jnp.int32(CHUNK)

    fill_zeros()
    zero_shared()
    plsc.subcore_barrier()

    def start_p(i, pb, semp):
        pltpu.async_copy(p_hbm.at[pl.ds(chunk_base(i), CHUNK)], pb, semp)

    def start_d(i, db, semd):
        pltpu.async_copy(dst1_hbm.at[pl.ds(chunk_base(i), CHUNK)], db, semd)

    start_p(jnp.int32(0), p0, semp0)
    start_d(jnp.int32(0), d0, semd0)
    start_p(jnp.int32(1), p1, semp1)
    start_d(jnp.int32(1), d1, semd1)

    def chunk_a(i, _):
        def do(pb, db, semp, semd):
            pltpu.make_async_copy(p_hbm.at[pl.ds(chunk_base(i), CHUNK)], pb,
                                  semp).wait()
            pltpu.make_async_copy(dst1_hbm.at[pl.ds(chunk_base(i), CHUNK)], db,
                                  semd).wait()
            pltpu.sync_copy(pb, acc_shared.at[db], add=True)

            @pl.when(i + 2 < NCHUNK)
            def _():
                start_p(i + jnp.int32(2), pb, semp)
                start_d(i + jnp.int32(2), db, semd)

        @pl.when(i % 2 == 0)
        def _():
            do(p0, d0, semp0, semd0)

        @pl.when(i % 2 == 1)
        def _():
            do(p1, d1, semp1, semd1)

        return jnp.int32(0)

    lax.fori_loop(jnp.int32(0), jnp.int32(NCHUNK), chunk_a, jnp.int32(0))
    plsc.subcore_barrier()
    export(den_hbm)
    plsc.subcore_barrier()

    fill_zeros()
    zero_shared()
    plsc.subcore_barrier()

    def start_sv(i, sb, vb, sems, semv):
        pltpu.sync_copy(src1_hbm.at[pl.ds(chunk_base(i), CHUNK)], sb)
        pltpu.async_copy(v_hbm.at[sb], vb, semv)

    start_p(jnp.int32(0), p0, semp0)
    start_d(jnp.int32(0), d0, semd0)
    start_sv(jnp.int32(0), s0, v0, sems0, semv0)
    start_p(jnp.int32(1), p1, semp1)
    start_d(jnp.int32(1), d1, semd1)
    start_sv(jnp.int32(1), s1, v1, sems1, semv1)

    def chunk_b(i, _):
        def do(pb, db, sb, vb, semp, semd, sems, semv):
            pltpu.make_async_copy(p_hbm.at[pl.ds(chunk_base(i), CHUNK)], pb,
                                  semp).wait()
            pltpu.make_async_copy(dst1_hbm.at[pl.ds(chunk_base(i), CHUNK)], db,
                                  semd).wait()
            pltpu.make_async_copy(v_hbm.at[sb], vb, semv).wait()

            def mrow(r, _):
                for k in range(HD // 16):
                    sl = pl.ds(k * 16, 16)
                    pb[r, sl] = pb[r, sl] * vb[r, sl]
                return jnp.int32(0)

            lax.fori_loop(jnp.int32(0), jnp.int32(CHUNK), mrow, jnp.int32(0))
            pltpu.sync_copy(pb, acc_shared.at[db], add=True)

            @pl.when(i + 2 < NCHUNK)
            def _():
                start_p(i + jnp.int32(2), pb, semp)
                start_d(i + jnp.int32(2), db, semd)
                start_sv(i + jnp.int32(2), sb, vb, sems, semv)

        @pl.when(i % 2 == 0)
        def _():
            do(p0, d0, s0, v0, semp0, semd0, sems0, semv0)

        @pl.when(i % 2 == 1)
        def _():
            do(p1, d1, s1, v1, semp1, semd1, sems1, semv1)

        return jnp.int32(0)

    lax.fori_loop(jnp.int32(0), jnp.int32(NCHUNK), chunk_b, jnp.int32(0))
    plsc.subcore_barrier()
    export(wv_hbm)


def _aggregate(P, V, src1, dst1):
    out = jax.ShapeDtypeStruct((2, NPAD, HD), jnp.float32)
    return pl.kernel(
        _agg_body,
        out_type=(out, out),
        mesh=_mesh,
        scratch_types=[
            pltpu.VMEM((CHUNK,), jnp.int32),
            pltpu.VMEM((CHUNK,), jnp.int32),
            pltpu.VMEM((CHUNK,), jnp.int32),
            pltpu.VMEM((CHUNK,), jnp.int32),
            pltpu.VMEM((CHUNK, HD), jnp.float32),
            pltpu.VMEM((CHUNK, HD), jnp.float32),
            pltpu.VMEM((CHUNK, HD), jnp.float32),
            pltpu.VMEM((CHUNK, HD), jnp.float32),
            pltpu.VMEM((EXPORT_ROWS, HD), jnp.float32),
            pltpu.VMEM_SHARED((NPAD, HD), jnp.float32),
            pltpu.SemaphoreType.DMA,
            pltpu.SemaphoreType.DMA,
            pltpu.SemaphoreType.DMA,
            pltpu.SemaphoreType.DMA,
            pltpu.SemaphoreType.DMA,
            pltpu.SemaphoreType.DMA,
            pltpu.SemaphoreType.DMA,
            pltpu.SemaphoreType.DMA,
        ],
    )(P, V, src1, dst1)


def _div_body(wv_ref, den_ref, out_ref):
    wv = wv_ref[0] + wv_ref[1]
    den = den_ref[0] + den_ref[1]
    out_ref[...] = wv / den


def _divide(wv_parts, den_parts):
    rows = 2000
    inspec = pl.BlockSpec((2, rows, HD), lambda i: (jnp.int32(0), i, jnp.int32(0)))
    return pl.pallas_call(
        _div_body,
        grid=(N_NODES // rows,),
        in_specs=[inspec, inspec],
        out_specs=pl.BlockSpec((rows, HD), lambda i: (i, jnp.int32(0))),
        out_shape=jax.ShapeDtypeStruct((N_NODES, HD), jnp.float32),
    )(wv_parts, den_parts)


@jax.jit
def kernel(h, e, edge_index, WQ, WK, WV, We):
    h = h.astype(jnp.float32)
    e = e.astype(jnp.float32)
    src2 = edge_index[0].astype(jnp.int32).reshape(NW, NCHUNK, CHUNK)
    dst2 = edge_index[1].astype(jnp.int32).reshape(NW, NCHUNK, CHUNK)

    Q, K, V = _qkv(h, WQ.astype(jnp.float32), WK.astype(jnp.float32),
                   WV.astype(jnp.float32))
    score = _score(K, Q, src2, dst2)
    e_out, P = _eout(e, We.astype(jnp.float32), score)
    den_parts, wv_parts = _aggregate(
        P, V, edge_index[0].astype(jnp.int32), edge_index[1].astype(jnp.int32))
    wv = _divide(wv_parts, den_parts)
    return (wv.reshape(N_NODES, NUM_HEADS, OUT_DIM).astype(jnp.float64),
            e_out.reshape(N_EDGES, NUM_HEADS, OUT_DIM).astype(jnp.float64))
